# baseline (speedup 1.0000x reference)
import functools

import jax
import jax.numpy as jnp
from jax import lax
from jax.experimental import pallas as pl
from jax.experimental.pallas import tpu as pltpu
from jax.experimental.pallas import tpu_sc as plsc

_N = 10000
_E = 320000
_D = 128
_H = 128
_B = 512
_EH = 32
_NT = 16
_NP = 10240
_RPT = _NP // _NT
_KC = 160
_GS = 16
_NG = _KC // _GS
_EP = _KC * 128 * _NT
_BC = _RPT // 128

_f32 = jnp.float32


def _mesh():
    return plsc.VectorSubcoreMesh(core_axis_name="c", subcore_axis_name="s")


def _deg_body(dst_hbm, bat_hbm, deg_out, cnt_out, accd, accc, idx_v, zb_v,
              ones_v):
    c = lax.axis_index("c")
    s = lax.axis_index("s")
    for i in range(8):
        ones_v[pl.ds(i * 16, 16)] = jnp.ones((16,), _f32)
    for i in range(_RPT // 16):
        zb_v[pl.ds(i * 16, 16)] = jnp.zeros((16,), _f32)
    pltpu.sync_copy(zb_v, accd.at[pl.ds(s * _RPT, _RPT)])

    @pl.when(s == 0)
    def _():
        pltpu.sync_copy(zb_v, accc)

    plsc.subcore_barrier()

    def grp(g, carry):
        pltpu.sync_copy(dst_hbm.at[c, s, pl.ds(g * _GS, _GS)], idx_v)
        for j in range(_GS):
            pltpu.sync_copy(ones_v, accd.at[idx_v.at[j]], add=True)
        return carry

    lax.fori_loop(0, _NG, grp, 0)

    pltpu.sync_copy(bat_hbm.at[c, s], idx_v.at[pl.ds(0, _BC)])
    for j in range(_BC):
        pltpu.sync_copy(ones_v, accc.at[idx_v.at[j]], add=True)

    plsc.subcore_barrier()
    pltpu.sync_copy(accd.at[pl.ds(s * _RPT, _RPT)],
                    deg_out.at[c, pl.ds(s * _RPT, _RPT)])

    @pl.when(s == 0)
    def _():
        pltpu.sync_copy(accc, cnt_out.at[c])


_deg_kernel = functools.partial(
    pl.kernel,
    out_type=[
        jax.ShapeDtypeStruct((2, _NP), _f32),
        jax.ShapeDtypeStruct((2, _RPT), _f32),
    ],
    mesh=_mesh(),
    scratch_types=[
        pltpu.VMEM_SHARED((_NP,), _f32),
        pltpu.VMEM_SHARED((_RPT,), _f32),
        pltpu.VMEM((_GS, 128), jnp.int32),
        pltpu.VMEM((_RPT,), _f32),
        pltpu.VMEM((128,), _f32),
    ],
)(_deg_body)


def _agg_body(hp_hbm, pk_hbm, src_hbm, dst_hbm, out_hbm, acc, sidx, didx,
              pk_a, pk_b, fbuf, gsem):
    c = lax.axis_index("c")
    s = lax.axis_index("s")
    r0 = s * _RPT
    pltpu.sync_copy(hp_hbm.at[pl.ds(c * _NP + r0, _RPT)], acc.at[pl.ds(r0, _RPT)])
    plsc.subcore_barrier()

    def grp(g, carry):
        pltpu.sync_copy(src_hbm.at[c, s, pl.ds(g * _GS, _GS)], sidx)
        pltpu.sync_copy(dst_hbm.at[c, s, pl.ds(g * _GS, _GS)], didx)
        gd = [None] * _GS
        gd[0] = pltpu.async_copy(pk_hbm.at[sidx.at[0]], pk_a, gsem)
        for j in range(_GS):
            pk = pk_a if j % 2 == 0 else pk_b
            npk = pk_b if j % 2 == 0 else pk_a
            gd[j].wait()
            if j + 1 < _GS:
                gd[j + 1] = pltpu.async_copy(pk_hbm.at[sidx.at[j + 1]], npk,
                                             gsem)

            def conv(i, carry2, pk=pk):
                for rr in range(4):
                    r = i * 4 + rr
                    for q in range(4):
                        w = pk[r, pl.ds(q * 16, 16)]
                        lo = plsc.bitcast(lax.shift_left(w, 16), _f32)
                        hi = plsc.bitcast(
                            jnp.bitwise_and(w, jnp.int32(-65536)), _f32)
                        fbuf[r, pl.ds(q * 16, 16)] = lo
                        fbuf[r, pl.ds(64 + q * 16, 16)] = hi
                return carry2

            lax.fori_loop(0, 32, conv, 0)
            pltpu.sync_copy(fbuf, acc.at[didx.at[j]], add=True)
        return carry

    lax.fori_loop(0, _NG, grp, 0)
    plsc.subcore_barrier()
    pltpu.sync_copy(acc.at[pl.ds(r0, _RPT)],
                    out_hbm.at[pl.ds(c * _NP + r0, _RPT)])


_agg_kernel = functools.partial(
    pl.kernel,
    out_type=jax.ShapeDtypeStruct((2 * _NP, _H), _f32),
    mesh=_mesh(),
    scratch_types=[
        pltpu.VMEM_SHARED((_NP, _H), _f32),
        pltpu.VMEM((_GS, 128), jnp.int32),
        pltpu.VMEM((_GS, 128), jnp.int32),
        pltpu.VMEM((128, 64), jnp.int32),
        pltpu.VMEM((128, 64), jnp.int32),
        pltpu.VMEM((128, _H), _f32),
        pltpu.SemaphoreType.DMA,
    ],
    compiler_params=pltpu.CompilerParams(use_tc_tiling_on_sc=False,
                                         needs_layout_passes=False),
)(_agg_body)


def _pack_bf16(h):
    hb = lax.bitcast_convert_type(h.astype(jnp.bfloat16),
                                  jnp.uint16).astype(jnp.uint32)
    t = (hb[:, 64:] << 16) | hb[:, :64]
    return lax.bitcast_convert_type(t, jnp.int32)


def _pool_body(x_hbm, bat_hbm, p_out, accp, bidx, rows, zrows):
    c = lax.axis_index("c")
    s = lax.axis_index("s")
    nzr = _RPT // _NT
    for r in range(nzr):
        for k in range(_H // 16):
            zrows[r, pl.ds(k * 16, 16)] = jnp.zeros((16,), _f32)
    pltpu.sync_copy(zrows, accp.at[pl.ds(s * nzr, nzr)])
    pltpu.sync_copy(bat_hbm.at[c, s], bidx)
    plsc.subcore_barrier()

    def chunk(j, carry):
        pltpu.sync_copy(x_hbm.at[pl.ds(c * _NP + s * _RPT + j * 128, 128)], rows)
        pltpu.sync_copy(rows, accp.at[bidx.at[j]], add=True)
        return carry

    lax.fori_loop(0, _BC, chunk, 0)
    plsc.subcore_barrier()
    pltpu.sync_copy(accp.at[pl.ds(s * nzr, nzr)], p_out.at[c, pl.ds(s * nzr, nzr)])


_pool_kernel = functools.partial(
    pl.kernel,
    out_type=jax.ShapeDtypeStruct((2, _RPT, _H), _f32),
    mesh=_mesh(),
    scratch_types=[
        pltpu.VMEM_SHARED((_RPT, _H), _f32),
        pltpu.VMEM((_BC, 128), jnp.int32),
        pltpu.VMEM((128, _H), _f32),
        pltpu.VMEM((_RPT // _NT, _H), _f32),
    ],
)(_pool_body)


_BLK = 256
_NBLK = 2 * _NP // _BLK


def _h1_body(x_ref, deg_ref, w_ref, o_ref):
    dinv = lax.rsqrt(deg_ref[...] + 1.0)
    o_ref[...] = jnp.dot(x_ref[...], w_ref[...],
                         preferred_element_type=_f32) * dinv


def _h1_call(x, deg, w):
    return pl.pallas_call(
        _h1_body,
        grid=(_NBLK,),
        in_specs=[
            pl.BlockSpec((_BLK, _D), lambda i: (i, 0)),
            pl.BlockSpec((_BLK, 1), lambda i: (i, 0)),
            pl.BlockSpec((_D, _H), lambda i: (0, 0)),
        ],
        out_specs=pl.BlockSpec((_BLK, _H), lambda i: (i, 0)),
        out_shape=jax.ShapeDtypeStruct((2 * _NP, _H), _f32),
    )(x, deg, w)


def _h2_body(s1_ref, deg_ref, w_ref, b_ref, o_ref):
    dinv = lax.rsqrt(deg_ref[...] + 1.0)
    x1 = jnp.maximum(s1_ref[...] * dinv + b_ref[...], 0.0)
    o_ref[...] = jnp.dot(x1, w_ref[...], preferred_element_type=_f32) * dinv


def _h2_call(s1, deg, w, b):
    return pl.pallas_call(
        _h2_body,
        grid=(_NBLK,),
        in_specs=[
            pl.BlockSpec((_BLK, _H), lambda i: (i, 0)),
            pl.BlockSpec((_BLK, 1), lambda i: (i, 0)),
            pl.BlockSpec((_H, _H), lambda i: (0, 0)),
            pl.BlockSpec((1, _H), lambda i: (0, 0)),
        ],
        out_specs=pl.BlockSpec((_BLK, _H), lambda i: (i, 0)),
        out_shape=jax.ShapeDtypeStruct((2 * _NP, _H), _f32),
    )(s1, deg, w, b)


def _x2_body(s2_ref, deg_ref, b_ref, o_ref):
    dinv = lax.rsqrt(deg_ref[...] + 1.0)
    o_ref[...] = jnp.maximum(s2_ref[...] * dinv + b_ref[...], 0.0)


def _x2_call(s2, deg, b):
    return pl.pallas_call(
        _x2_body,
        grid=(_NBLK,),
        in_specs=[
            pl.BlockSpec((_BLK, _H), lambda i: (i, 0)),
            pl.BlockSpec((_BLK, 1), lambda i: (i, 0)),
            pl.BlockSpec((1, _H), lambda i: (0, 0)),
        ],
        out_specs=pl.BlockSpec((_BLK, _H), lambda i: (i, 0)),
        out_shape=jax.ShapeDtypeStruct((2 * _NP, _H), _f32),
    )(s2, deg, b)


def _sys_body(p_ref, cnt_ref, ef_ref, wp_ref, bp_ref, we1_ref, be1_ref,
              we2_ref, bmat_ref, wroot_ref, bnn_ref, wih_ref, whh_ref,
              bih_ref, bhh_ref, wc1_ref, bc1_ref, wc2_ref, bc2_ref, wc3_ref,
              bc3_ref, o_ref):
    relu = lambda v: jnp.maximum(v, 0.0)
    dot = lambda a, b: jnp.dot(a, b, preferred_element_type=_f32)
    xg = p_ref[...] / jnp.maximum(cnt_ref[...], 1.0)
    nf = relu(dot(xg, wp_ref[...]) + bp_ref[...])
    eact = relu(ef_ref[...] * we1_ref[...] + be1_ref[...])
    we2 = we2_ref[...]
    bmat = bmat_ref[...]
    mb = dot(nf, bmat)
    m1 = mb
    m2 = mb
    for k in range(_EH):
        uk = dot(nf, we2[k * _H:(k + 1) * _H, :])
        m1 = m1 + eact[0:2 * _B, k:k + 1] * uk
        m2 = m2 + eact[2 * _B:4 * _B, k:k + 1] * uk
    aggr = jnp.concatenate(
        [m1[_B:2 * _B] + m2[0:_B], m1[0:_B] + m2[_B:2 * _B]], axis=0)
    m = relu(dot(nf, wroot_ref[...]) + aggr + bnn_ref[...])
    gi = dot(m, wih_ref[...]) + bih_ref[...]
    gh = dot(nf, whh_ref[...]) + bhh_ref[...]
    r = jax.nn.sigmoid(gi[:, 0:_H] + gh[:, 0:_H])
    z = jax.nn.sigmoid(gi[:, _H:2 * _H] + gh[:, _H:2 * _H])
    nn_ = jnp.tanh(gi[:, 2 * _H:3 * _H] + r * gh[:, 2 * _H:3 * _H])
    xgo = (1.0 - z) * nn_ + z * nf
    xgc = jnp.concatenate([xgo[:_B], xgo[_B:]], axis=1)
    out = relu(dot(xgc, wc1_ref[...]) + bc1_ref[...])
    out = relu(dot(out, wc2_ref[...]) + bc2_ref[...])
    o_ref[...] = dot(out, wc3_ref[...]) + bc3_ref[...]


def _sys_call(p, cnt, ef, wp, bp, we1, be1, we2r, bmat, wroot, bnn, wih, whh,
              bih, bhh, wc1, bc1, wc2, bc2, wc3, bc3):
    return pl.pallas_call(
        _sys_body,
        out_shape=jax.ShapeDtypeStruct((_B, 1), _f32),
    )(p, cnt, ef, wp, bp, we1, be1, we2r, bmat, wroot, bnn, wih, whh, bih,
      bhh, wc1, bc1, wc2, bc2, wc3, bc3)


def kernel(solvent_x, solvent_edge_index, solvent_batch, solvent_inter_hb,
           solvent_y, solute_x, solute_edge_index, solute_batch,
           solute_inter_hb, W1, b1, W2, b2, Wp, bp, We1, be1, We2, be2,
           Wroot, bnn, Wih, Whh, bih, bhh, Wc1, bc1, Wc2, bc2, Wc3, bc3):
    Xp = jnp.zeros((2, _NP, _D), _f32)
    Xp = Xp.at[0, :_N].set(solvent_x).at[1, :_N].set(solute_x)
    Xp = Xp.reshape(2 * _NP, _D)

    def pad_edges(ei, c):
        src = jnp.full((_EP,), c * _NP, jnp.int32).at[:_E].set(ei[0] + c * _NP)
        dst = jnp.full((_EP,), _N, jnp.int32).at[:_E].set(ei[1])
        return src, dst

    s0, d0 = pad_edges(solvent_edge_index, 0)
    s1e, d1e = pad_edges(solute_edge_index, 1)
    src_g = jnp.stack([s0, s1e]).reshape(2, _NT, _KC, 128)
    dst_l = jnp.stack([d0, d1e]).reshape(2, _NT, _KC, 128)

    bat = jnp.full((2, _NP), _B, jnp.int32)
    bat = bat.at[0, :_N].set(solvent_batch).at[1, :_N].set(solute_batch)
    bat4 = bat.reshape(2, _NT, _BC, 128)

    deg2, cnt2 = _deg_kernel(dst_l, bat4)
    deg = deg2.reshape(2 * _NP, 1)
    cnt = cnt2[:, :_B].reshape(2 * _B, 1)

    h1p = _h1_call(Xp, deg, W1)

    S1 = _agg_kernel(h1p, _pack_bf16(h1p), src_g, dst_l)

    h2p = _h2_call(S1, deg, W2, b1.reshape(1, _H))

    S2 = _agg_kernel(h2p, _pack_bf16(h2p), src_g, dst_l)

    x2 = _x2_call(S2, deg, b2.reshape(1, _H))

    P2 = _pool_kernel(x2, bat4)
    P = P2[:, :_B].reshape(2 * _B, _H)

    ef = jnp.concatenate(
        [solvent_inter_hb, solvent_inter_hb, solvent_inter_hb,
         solute_inter_hb])[:, None]
    out = _sys_call(P, cnt, ef, Wp, bp.reshape(1, _H), We1, be1.reshape(1, _EH),
                    We2.reshape(_EH * _H, _H), be2.reshape(_H, _H), Wroot,
                    bnn.reshape(1, _H), Wih, Whh, bih.reshape(1, 3 * _H),
                    bhh.reshape(1, 3 * _H), Wc1, bc1.reshape(1, _H), Wc2,
                    bc2.reshape(1, _H), Wc3, bc3.reshape(1, 1))
    return out


# speedup vs baseline: 1.3278x; 1.3278x over previous
import functools

import jax
import jax.numpy as jnp
from jax import lax
from jax.experimental import pallas as pl
from jax.experimental.pallas import tpu as pltpu
from jax.experimental.pallas import tpu_sc as plsc

_N = 10000
_E = 320000
_D = 128
_H = 128
_B = 512
_EH = 32
_NT = 16
_NP = 10240
_RPT = _NP // _NT
_KC = 160
_GS = 16
_NG = _KC // _GS
_EP = _KC * 128 * _NT
_BC = _RPT // 128

_f32 = jnp.float32


def _mesh():
    return plsc.VectorSubcoreMesh(core_axis_name="c", subcore_axis_name="s")


def _deg_body(dst_hbm, bat_hbm, deg_out, cnt_out, accd, accc, idx_v, zb_v,
              ones_v):
    c = lax.axis_index("c")
    s = lax.axis_index("s")
    for i in range(8):
        ones_v[pl.ds(i * 16, 16)] = jnp.ones((16,), _f32)
    for i in range(_RPT // 16):
        zb_v[pl.ds(i * 16, 16)] = jnp.zeros((16,), _f32)
    pltpu.sync_copy(zb_v, accd.at[pl.ds(s * _RPT, _RPT)])

    @pl.when(s == 0)
    def _():
        pltpu.sync_copy(zb_v, accc)

    plsc.subcore_barrier()

    def grp(g, carry):
        pltpu.sync_copy(dst_hbm.at[c, s, pl.ds(g * _GS, _GS)], idx_v)
        for j in range(_GS):
            pltpu.sync_copy(ones_v, accd.at[idx_v.at[j]], add=True)
        return carry

    lax.fori_loop(0, _NG, grp, 0)

    pltpu.sync_copy(bat_hbm.at[c, s], idx_v.at[pl.ds(0, _BC)])
    for j in range(_BC):
        pltpu.sync_copy(ones_v, accc.at[idx_v.at[j]], add=True)

    plsc.subcore_barrier()
    pltpu.sync_copy(accd.at[pl.ds(s * _RPT, _RPT)],
                    deg_out.at[c, pl.ds(s * _RPT, _RPT)])

    @pl.when(s == 0)
    def _():
        pltpu.sync_copy(accc, cnt_out.at[c])


_deg_kernel = functools.partial(
    pl.kernel,
    out_type=[
        jax.ShapeDtypeStruct((2, _NP), _f32),
        jax.ShapeDtypeStruct((2, _RPT), _f32),
    ],
    mesh=_mesh(),
    scratch_types=[
        pltpu.VMEM_SHARED((_NP,), _f32),
        pltpu.VMEM_SHARED((_RPT,), _f32),
        pltpu.VMEM((_GS, 128), jnp.int32),
        pltpu.VMEM((_RPT,), _f32),
        pltpu.VMEM((128,), _f32),
    ],
)(_deg_body)


def _agg_body(hp_hbm, pk_hbm, src_hbm, dst_hbm, out_hbm, acc, sidx, didx,
              pk_a, pk_b, fbuf, gsem):
    c = lax.axis_index("c")
    s = lax.axis_index("s")
    r0 = s * _RPT
    pltpu.sync_copy(hp_hbm.at[pl.ds(c * _NP + r0, _RPT)], acc.at[pl.ds(r0, _RPT)])
    plsc.subcore_barrier()

    def grp(g, carry):
        pltpu.sync_copy(src_hbm.at[c, s, pl.ds(g * _GS, _GS)], sidx)
        pltpu.sync_copy(dst_hbm.at[c, s, pl.ds(g * _GS, _GS)], didx)
        gd = [None] * _GS
        gd[0] = pltpu.async_copy(pk_hbm.at[sidx.at[0]], pk_a, gsem)
        for j in range(_GS):
            pk = pk_a if j % 2 == 0 else pk_b
            npk = pk_b if j % 2 == 0 else pk_a
            gd[j].wait()
            if j + 1 < _GS:
                gd[j + 1] = pltpu.async_copy(pk_hbm.at[sidx.at[j + 1]], npk,
                                             gsem)

            def conv(r, pk=pk):
                ws = [pk[r, pl.ds(q * 16, 16)] for q in range(4)]
                for q in range(4):
                    lo = plsc.bitcast(lax.shift_left(ws[q], 16), _f32)
                    hi = plsc.bitcast(
                        jnp.bitwise_and(ws[q], jnp.int32(-65536)), _f32)
                    fbuf[r, pl.ds(q * 16, 16)] = lo
                    fbuf[r, pl.ds(64 + q * 16, 16)] = hi

            plsc.parallel_loop(0, 128, 1, unroll=8)(conv)
            pltpu.sync_copy(fbuf, acc.at[didx.at[j]], add=True)
        return carry

    lax.fori_loop(0, _NG, grp, 0)
    plsc.subcore_barrier()
    pltpu.sync_copy(acc.at[pl.ds(r0, _RPT)],
                    out_hbm.at[pl.ds(c * _NP + r0, _RPT)])


_agg_kernel = functools.partial(
    pl.kernel,
    out_type=jax.ShapeDtypeStruct((2 * _NP, _H), _f32),
    mesh=_mesh(),
    scratch_types=[
        pltpu.VMEM_SHARED((_NP, _H), _f32),
        pltpu.VMEM((_GS, 128), jnp.int32),
        pltpu.VMEM((_GS, 128), jnp.int32),
        pltpu.VMEM((128, 64), jnp.int32),
        pltpu.VMEM((128, 64), jnp.int32),
        pltpu.VMEM((128, _H), _f32),
        pltpu.SemaphoreType.DMA,
    ],
    compiler_params=pltpu.CompilerParams(use_tc_tiling_on_sc=False,
                                         needs_layout_passes=False),
)(_agg_body)


def _pack_bf16(h):
    hb = lax.bitcast_convert_type(h.astype(jnp.bfloat16),
                                  jnp.uint16).astype(jnp.uint32)
    t = (hb[:, 64:] << 16) | hb[:, :64]
    return lax.bitcast_convert_type(t, jnp.int32)


def _pool_body(x_hbm, bat_hbm, p_out, accp, bidx, rows, zrows):
    c = lax.axis_index("c")
    s = lax.axis_index("s")
    nzr = _RPT // _NT
    for r in range(nzr):
        for k in range(_H // 16):
            zrows[r, pl.ds(k * 16, 16)] = jnp.zeros((16,), _f32)
    pltpu.sync_copy(zrows, accp.at[pl.ds(s * nzr, nzr)])
    pltpu.sync_copy(bat_hbm.at[c, s], bidx)
    plsc.subcore_barrier()

    def chunk(j, carry):
        pltpu.sync_copy(x_hbm.at[pl.ds(c * _NP + s * _RPT + j * 128, 128)], rows)
        pltpu.sync_copy(rows, accp.at[bidx.at[j]], add=True)
        return carry

    lax.fori_loop(0, _BC, chunk, 0)
    plsc.subcore_barrier()
    pltpu.sync_copy(accp.at[pl.ds(s * nzr, nzr)], p_out.at[c, pl.ds(s * nzr, nzr)])


_pool_kernel = functools.partial(
    pl.kernel,
    out_type=jax.ShapeDtypeStruct((2, _RPT, _H), _f32),
    mesh=_mesh(),
    scratch_types=[
        pltpu.VMEM_SHARED((_RPT, _H), _f32),
        pltpu.VMEM((_BC, 128), jnp.int32),
        pltpu.VMEM((128, _H), _f32),
        pltpu.VMEM((_RPT // _NT, _H), _f32),
    ],
)(_pool_body)


_BLK = 256
_NBLK = 2 * _NP // _BLK


def _h1_body(x_ref, deg_ref, w_ref, o_ref):
    dinv = lax.rsqrt(deg_ref[...] + 1.0)
    o_ref[...] = jnp.dot(x_ref[...], w_ref[...],
                         preferred_element_type=_f32) * dinv


def _h1_call(x, deg, w):
    return pl.pallas_call(
        _h1_body,
        grid=(_NBLK,),
        in_specs=[
            pl.BlockSpec((_BLK, _D), lambda i: (i, 0)),
            pl.BlockSpec((_BLK, 1), lambda i: (i, 0)),
            pl.BlockSpec((_D, _H), lambda i: (0, 0)),
        ],
        out_specs=pl.BlockSpec((_BLK, _H), lambda i: (i, 0)),
        out_shape=jax.ShapeDtypeStruct((2 * _NP, _H), _f32),
    )(x, deg, w)


def _h2_body(s1_ref, deg_ref, w_ref, b_ref, o_ref):
    dinv = lax.rsqrt(deg_ref[...] + 1.0)
    x1 = jnp.maximum(s1_ref[...] * dinv + b_ref[...], 0.0)
    o_ref[...] = jnp.dot(x1, w_ref[...], preferred_element_type=_f32) * dinv


def _h2_call(s1, deg, w, b):
    return pl.pallas_call(
        _h2_body,
        grid=(_NBLK,),
        in_specs=[
            pl.BlockSpec((_BLK, _H), lambda i: (i, 0)),
            pl.BlockSpec((_BLK, 1), lambda i: (i, 0)),
            pl.BlockSpec((_H, _H), lambda i: (0, 0)),
            pl.BlockSpec((1, _H), lambda i: (0, 0)),
        ],
        out_specs=pl.BlockSpec((_BLK, _H), lambda i: (i, 0)),
        out_shape=jax.ShapeDtypeStruct((2 * _NP, _H), _f32),
    )(s1, deg, w, b)


def _x2_body(s2_ref, deg_ref, b_ref, o_ref):
    dinv = lax.rsqrt(deg_ref[...] + 1.0)
    o_ref[...] = jnp.maximum(s2_ref[...] * dinv + b_ref[...], 0.0)


def _x2_call(s2, deg, b):
    return pl.pallas_call(
        _x2_body,
        grid=(_NBLK,),
        in_specs=[
            pl.BlockSpec((_BLK, _H), lambda i: (i, 0)),
            pl.BlockSpec((_BLK, 1), lambda i: (i, 0)),
            pl.BlockSpec((1, _H), lambda i: (0, 0)),
        ],
        out_specs=pl.BlockSpec((_BLK, _H), lambda i: (i, 0)),
        out_shape=jax.ShapeDtypeStruct((2 * _NP, _H), _f32),
    )(s2, deg, b)


def _sys_body(p_ref, cnt_ref, ef_ref, wp_ref, bp_ref, we1_ref, be1_ref,
              we2_ref, bmat_ref, wroot_ref, bnn_ref, wih_ref, whh_ref,
              bih_ref, bhh_ref, wc1_ref, bc1_ref, wc2_ref, bc2_ref, wc3_ref,
              bc3_ref, o_ref):
    relu = lambda v: jnp.maximum(v, 0.0)
    dot = lambda a, b: jnp.dot(a, b, preferred_element_type=_f32)
    xg = p_ref[...] / jnp.maximum(cnt_ref[...], 1.0)
    nf = relu(dot(xg, wp_ref[...]) + bp_ref[...])
    eact = relu(ef_ref[...] * we1_ref[...] + be1_ref[...])
    we2 = we2_ref[...]
    bmat = bmat_ref[...]
    mb = dot(nf, bmat)
    m1 = mb
    m2 = mb
    for k in range(_EH):
        uk = dot(nf, we2[k * _H:(k + 1) * _H, :])
        m1 = m1 + eact[0:2 * _B, k:k + 1] * uk
        m2 = m2 + eact[2 * _B:4 * _B, k:k + 1] * uk
    aggr = jnp.concatenate(
        [m1[_B:2 * _B] + m2[0:_B], m1[0:_B] + m2[_B:2 * _B]], axis=0)
    m = relu(dot(nf, wroot_ref[...]) + aggr + bnn_ref[...])
    gi = dot(m, wih_ref[...]) + bih_ref[...]
    gh = dot(nf, whh_ref[...]) + bhh_ref[...]
    r = jax.nn.sigmoid(gi[:, 0:_H] + gh[:, 0:_H])
    z = jax.nn.sigmoid(gi[:, _H:2 * _H] + gh[:, _H:2 * _H])
    nn_ = jnp.tanh(gi[:, 2 * _H:3 * _H] + r * gh[:, 2 * _H:3 * _H])
    xgo = (1.0 - z) * nn_ + z * nf
    xgc = jnp.concatenate([xgo[:_B], xgo[_B:]], axis=1)
    out = relu(dot(xgc, wc1_ref[...]) + bc1_ref[...])
    out = relu(dot(out, wc2_ref[...]) + bc2_ref[...])
    o_ref[...] = dot(out, wc3_ref[...]) + bc3_ref[...]


def _sys_call(p, cnt, ef, wp, bp, we1, be1, we2r, bmat, wroot, bnn, wih, whh,
              bih, bhh, wc1, bc1, wc2, bc2, wc3, bc3):
    return pl.pallas_call(
        _sys_body,
        out_shape=jax.ShapeDtypeStruct((_B, 1), _f32),
    )(p, cnt, ef, wp, bp, we1, be1, we2r, bmat, wroot, bnn, wih, whh, bih,
      bhh, wc1, bc1, wc2, bc2, wc3, bc3)


def kernel(solvent_x, solvent_edge_index, solvent_batch, solvent_inter_hb,
           solvent_y, solute_x, solute_edge_index, solute_batch,
           solute_inter_hb, W1, b1, W2, b2, Wp, bp, We1, be1, We2, be2,
           Wroot, bnn, Wih, Whh, bih, bhh, Wc1, bc1, Wc2, bc2, Wc3, bc3):
    Xp = jnp.zeros((2, _NP, _D), _f32)
    Xp = Xp.at[0, :_N].set(solvent_x).at[1, :_N].set(solute_x)
    Xp = Xp.reshape(2 * _NP, _D)

    def pad_edges(ei, c):
        src = jnp.full((_EP,), c * _NP, jnp.int32).at[:_E].set(ei[0] + c * _NP)
        dst = jnp.full((_EP,), _N, jnp.int32).at[:_E].set(ei[1])
        return src, dst

    s0, d0 = pad_edges(solvent_edge_index, 0)
    s1e, d1e = pad_edges(solute_edge_index, 1)
    src_g = jnp.stack([s0, s1e]).reshape(2, _NT, _KC, 128)
    dst_l = jnp.stack([d0, d1e]).reshape(2, _NT, _KC, 128)

    bat = jnp.full((2, _NP), _B, jnp.int32)
    bat = bat.at[0, :_N].set(solvent_batch).at[1, :_N].set(solute_batch)
    bat4 = bat.reshape(2, _NT, _BC, 128)

    deg2, cnt2 = _deg_kernel(dst_l, bat4)
    deg = deg2.reshape(2 * _NP, 1)
    cnt = cnt2[:, :_B].reshape(2 * _B, 1)

    h1p = _h1_call(Xp, deg, W1)

    S1 = _agg_kernel(h1p, _pack_bf16(h1p), src_g, dst_l)

    h2p = _h2_call(S1, deg, W2, b1.reshape(1, _H))

    S2 = _agg_kernel(h2p, _pack_bf16(h2p), src_g, dst_l)

    x2 = _x2_call(S2, deg, b2.reshape(1, _H))

    P2 = _pool_kernel(x2, bat4)
    P = P2[:, :_B].reshape(2 * _B, _H)

    ef = jnp.concatenate(
        [solvent_inter_hb, solvent_inter_hb, solvent_inter_hb,
         solute_inter_hb])[:, None]
    out = _sys_call(P, cnt, ef, Wp, bp.reshape(1, _H), We1, be1.reshape(1, _EH),
                    We2.reshape(_EH * _H, _H), be2.reshape(_H, _H), Wroot,
                    bnn.reshape(1, _H), Wih, Whh, bih.reshape(1, 3 * _H),
                    bhh.reshape(1, 3 * _H), Wc1, bc1.reshape(1, _H), Wc2,
                    bc2.reshape(1, _H), Wc3, bc3.reshape(1, 1))
    return out
